# Initial kernel scaffold; baseline (speedup 1.0000x reference)
import jax
import jax.numpy as jnp
from jax.experimental import pallas as pl


def kernel(x, router):
    raise NotImplementedError("write your pallas kernel here")


# speedup vs baseline: 1.0259x; 1.0259x over previous
import jax
import jax.numpy as jnp
from jax import lax
from jax.experimental import pallas as pl
from jax.experimental.pallas import tpu as pltpu
from jax.experimental.pallas import tpu_sc as plsc

_BATCH = 4
_SEQ = 4096
_LANES = 128
_ROWS = (_BATCH * _SEQ) // _LANES
_NW = 32
_RPW = _ROWS // _NW


def _router_gather(router_hbm, x_hbm, out_hbm, idx_v, val_v, sem):
    wid = lax.axis_index("s") * 2 + lax.axis_index("c")
    base = wid * _RPW
    pltpu.sync_copy(x_hbm.at[pl.ds(base, _RPW)], idx_v)
    copies = [
        pltpu.async_copy(router_hbm.at[idx_v.at[j]], val_v.at[j], sem)
        for j in range(_RPW)
    ]
    for c in copies:
        c.wait()
    pltpu.sync_copy(val_v, out_hbm.at[pl.ds(base, _RPW)])


def kernel(x, router):
    x2 = x.reshape(_ROWS, _LANES).astype(jnp.int32)
    router = router.astype(jnp.int32)
    mesh = plsc.VectorSubcoreMesh(core_axis_name="c", subcore_axis_name="s")
    out = pl.kernel(
        _router_gather,
        out_type=jax.ShapeDtypeStruct((_ROWS, _LANES), jnp.int32),
        mesh=mesh,
        scratch_types=[
            pltpu.VMEM((_RPW, _LANES), jnp.int32),
            pltpu.VMEM((_RPW, _LANES), jnp.int32),
            pltpu.SemaphoreType.DMA,
        ],
    )(router, x2)
    return out.reshape(_BATCH, _SEQ)


# speedup vs baseline: 1.0356x; 1.0094x over previous
import jax
import jax.numpy as jnp
from jax import lax
from jax.experimental import pallas as pl
from jax.experimental.pallas import tpu as pltpu
from jax.experimental.pallas import tpu_sc as plsc

_BATCH = 4
_SEQ = 4096
_NW = 32
_TPW = (_BATCH * _SEQ) // _NW


def _router_gather(router_hbm, x_hbm, out_hbm, idx_v, val_v, sem):
    wid = lax.axis_index("s") * 2 + lax.axis_index("c")
    base = wid * _TPW
    pltpu.sync_copy(x_hbm.at[pl.ds(base, _TPW)], idx_v)
    pltpu.async_copy(router_hbm.at[idx_v], val_v, sem).wait()
    pltpu.sync_copy(val_v, out_hbm.at[pl.ds(base, _TPW)])


def kernel(x, router):
    x1 = x.reshape(_BATCH * _SEQ).astype(jnp.int32)
    router = router.astype(jnp.int32)
    mesh = plsc.VectorSubcoreMesh(core_axis_name="c", subcore_axis_name="s")
    out = pl.kernel(
        _router_gather,
        out_type=jax.ShapeDtypeStruct((_BATCH * _SEQ,), jnp.int32),
        mesh=mesh,
        scratch_types=[
            pltpu.VMEM((_TPW,), jnp.int32),
            pltpu.VMEM((_TPW,), jnp.int32),
            pltpu.SemaphoreType.DMA,
        ],
    )(router, x1)
    return out.reshape(_BATCH, _SEQ)
